# baseline (speedup 1.0000x reference)
import functools

import jax
import jax.numpy as jnp
from jax import lax
from jax.experimental import pallas as pl
from jax.experimental.pallas import tpu as pltpu
from jax.experimental.pallas import tpu_sc as plsc

VOCAB = 100000
D = 16
BATCH = 1024
SEQ = 200

NC = 2
NS = 16
NW = NC * NS
ROWS_PER_W = BATCH // NW
TOK_PER_W = ROWS_PER_W * SEQ

CH0 = 128
CH1 = SEQ - CH0
NBUF = 16


def _make_kernel():
    mesh = plsc.VectorSubcoreMesh(
        core_axis_name="c", subcore_axis_name="s", num_cores=NC, num_subcores=NS)

    @functools.partial(
        pl.kernel,
        mesh=mesh,
        out_type=jax.ShapeDtypeStruct((BATCH, D), jnp.float32),
        compiler_params=pltpu.CompilerParams(use_tc_tiling_on_sc=False),
        scratch_types=[
            pltpu.VMEM((TOK_PER_W,), jnp.int32),
            pltpu.VMEM((NBUF, SEQ, D), jnp.float32),
            pltpu.VMEM((ROWS_PER_W, D), jnp.float32),
            pltpu.VMEM((D,), jnp.float32),
            pltpu.SemaphoreType.DMA((NBUF,)),
        ],
    )
    def k(tok_hbm, w_hbm, b_hbm, out_hbm, tok_v, rows_v, out_v, b_v, sem):
        sid = lax.axis_index("s")
        wid = sid * NC + lax.axis_index("c")
        base = wid * ROWS_PER_W
        pltpu.sync_copy(tok_hbm.at[pl.ds(base * SEQ, TOK_PER_W)], tok_v)
        pltpu.sync_copy(b_hbm, b_v)

        SPLITS = ((0, CH0), (CH0, CH1))

        def fire(r, slot):
            for off, n in SPLITS:
                pltpu.make_async_copy(
                    w_hbm.at[tok_v.at[pl.ds(r * SEQ + off, n)]],
                    rows_v.at[slot, pl.ds(off, n)], sem.at[slot]).start()

        def drain(r, slot):
            for off, n in SPLITS:
                pltpu.make_async_copy(
                    w_hbm.at[tok_v.at[pl.ds(r * SEQ + off, n)]],
                    rows_v.at[slot, pl.ds(off, n)], sem.at[slot]).wait()

        for r0 in range(NBUF):
            fire(r0, r0)
        UNROLL = 8

        def row_body(r, carry):
            slot = lax.rem(r, NBUF)
            drain(r, slot)

            @pl.when(r + NBUF < ROWS_PER_W)
            def _():
                fire(r + NBUF, slot)

            def acc_body(j, accs):
                jj = j * UNROLL
                return tuple(
                    accs[u] + rows_v[slot, jj + u] for u in range(UNROLL))

            zero = jnp.zeros((D,), jnp.float32)
            accs = (b_v[...],) + (zero,) * (UNROLL - 1)
            accs = lax.fori_loop(0, SEQ // UNROLL, acc_body, accs)
            a = accs[0]
            for u in range(1, UNROLL):
                a = a + accs[u]
            out_v[r] = a
            return carry

        lax.fori_loop(0, ROWS_PER_W, row_body, 0)
        pltpu.sync_copy(out_v, out_hbm.at[pl.ds(base, ROWS_PER_W)])

    return k


_kernel = _make_kernel()


def kernel(tokens, W, b):
    return _kernel(tokens.reshape(-1).astype(jnp.int32), W, b)


# speedup vs baseline: 1.0028x; 1.0028x over previous
import functools

import jax
import jax.numpy as jnp
from jax import lax
from jax.experimental import pallas as pl
from jax.experimental.pallas import tpu as pltpu
from jax.experimental.pallas import tpu_sc as plsc

VOCAB = 100000
D = 16
BATCH = 1024
SEQ = 200

NC = 2
NS = 16
NW = NC * NS
ROWS_PER_W = BATCH // NW
TOK_PER_W = ROWS_PER_W * SEQ

CH0 = 128
CH1 = SEQ - CH0
NBUF = 16


def _make_kernel():
    mesh = plsc.VectorSubcoreMesh(
        core_axis_name="c", subcore_axis_name="s", num_cores=NC, num_subcores=NS)

    @functools.partial(
        pl.kernel,
        mesh=mesh,
        out_type=jax.ShapeDtypeStruct((BATCH, D), jnp.float32),
        compiler_params=pltpu.CompilerParams(use_tc_tiling_on_sc=False),
        scratch_types=[
            pltpu.VMEM((TOK_PER_W,), jnp.int32),
            pltpu.VMEM((NBUF, SEQ, D), jnp.float32),
            pltpu.VMEM((ROWS_PER_W, D), jnp.float32),
            pltpu.VMEM((D,), jnp.float32),
            pltpu.SemaphoreType.DMA((NBUF,)),
            pltpu.SemaphoreType.DMA,
        ],
    )
    def k(tok_hbm, w_hbm, b_hbm, out_hbm, tok_v, rows_v, out_v, b_v, sem,
          sem2):
        sid = lax.axis_index("s")
        wid = sid * NC + lax.axis_index("c")
        base = wid * ROWS_PER_W
        half = TOK_PER_W // 2
        pltpu.sync_copy(tok_hbm.at[pl.ds(base * SEQ, half)],
                        tok_v.at[pl.ds(0, half)])
        tail_cp = pltpu.make_async_copy(
            tok_hbm.at[pl.ds(base * SEQ + half, half)],
            tok_v.at[pl.ds(half, half)], sem2)
        tail_cp.start()
        bias_cp = pltpu.make_async_copy(b_hbm, b_v, sem2)

        SPLITS = ((0, CH0), (CH0, CH1))

        def fire(r, slot):
            for off, n in SPLITS:
                pltpu.make_async_copy(
                    w_hbm.at[tok_v.at[pl.ds(r * SEQ + off, n)]],
                    rows_v.at[slot, pl.ds(off, n)], sem.at[slot]).start()

        def drain(r, slot):
            for off, n in SPLITS:
                pltpu.make_async_copy(
                    w_hbm.at[tok_v.at[pl.ds(r * SEQ + off, n)]],
                    rows_v.at[slot, pl.ds(off, n)], sem.at[slot]).wait()

        for r0 in range(NBUF):
            fire(r0, r0)
        tail_cp.wait()
        bias_cp.start()
        bias_cp.wait()
        UNROLL = 8

        def row_body(r, carry):
            slot = lax.rem(r, NBUF)
            drain(r, slot)

            @pl.when(r + NBUF < ROWS_PER_W)
            def _():
                fire(r + NBUF, slot)

            def acc_body(j, accs):
                jj = j * UNROLL
                return tuple(
                    accs[u] + rows_v[slot, jj + u] for u in range(UNROLL))

            zero = jnp.zeros((D,), jnp.float32)
            accs = (b_v[...],) + (zero,) * (UNROLL - 1)
            accs = lax.fori_loop(0, SEQ // UNROLL, acc_body, accs)
            a = accs[0]
            for u in range(1, UNROLL):
                a = a + accs[u]
            out_v[r] = a
            return carry

        lax.fori_loop(0, ROWS_PER_W, row_body, 0)
        pltpu.sync_copy(out_v, out_hbm.at[pl.ds(base, ROWS_PER_W)])

    return k


_kernel = _make_kernel()


def kernel(tokens, W, b):
    return _kernel(tokens.reshape(-1).astype(jnp.int32), W, b)
